# baseline (speedup 1.0000x reference)
import functools

import jax
import jax.numpy as jnp
from jax import lax
from jax.experimental import pallas as pl
from jax.experimental.pallas import tpu as pltpu
from jax.experimental.pallas import tpu_sc as plsc

_T = 200
_B = 16384
_D = 32
_UB = 256
_NBUF = 2


def _gather_fn(v):
    info = plsc.get_sparse_core_info()
    nc, ns = info.num_cores, info.num_subcores
    nw = nc * ns
    nunits = _T * (_B // _UB)
    per_w = nunits // nw
    ppt = _B // _UB
    assert nunits % nw == 0

    mesh = plsc.VectorSubcoreMesh(core_axis_name="c", subcore_axis_name="s")

    @functools.partial(
        pl.kernel,
        out_type=jax.ShapeDtypeStruct((_T, _D // 8, _B // 128, 8, 128),
                                      jnp.float32),
        mesh=mesh,
        scratch_types=[
            pltpu.VMEM((_NBUF, _UB), jnp.int32),
            pltpu.VMEM((_NBUF, _UB, _D), jnp.float32),
            pltpu.VMEM((_NBUF, _D // 8, _UB // 128, 8, 128), jnp.float32),
            [pltpu.SemaphoreType.DMA] * _NBUF,
            [pltpu.SemaphoreType.DMA] * _NBUF,
            [pltpu.SemaphoreType.DMA] * _NBUF,
        ],
        compiler_params=pltpu.CompilerParams(use_tc_tiling_on_sc=False,
                                             needs_layout_passes=False),
    )
    def run(idx_hbm, table_hbm, out_hbm, idx_v, rows_v, tile_v,
            isems, gsems, osems):
        wid = lax.axis_index("s") * nc + lax.axis_index("c")
        u0 = wid * per_w
        iota = lax.iota(jnp.int32, 16)

        def unit_tp(u):
            t = u // ppt
            p = u - t * ppt
            return t, p

        def start_idx(s, u):
            t, p = unit_tp(u)
            off = pl.multiple_of(t * _B + p * _UB, 8)
            pltpu.async_copy(idx_hbm.at[pl.ds(off, _UB)], idx_v.at[s],
                             isems[s])

        def wait_idx(s):
            pltpu.make_async_copy(idx_hbm.at[pl.ds(0, _UB)], idx_v.at[s],
                                  isems[s]).wait()

        def start_gather(s):
            pltpu.async_copy(table_hbm.at[idx_v.at[s]], rows_v.at[s],
                             gsems[s])

        def wait_gather(s):
            pltpu.make_async_copy(table_hbm.at[idx_v.at[s]], rows_v.at[s],
                                  gsems[s]).wait()

        def start_out(s, u):
            t, p = unit_tp(u)
            for cb in range(_D // 8):
                pltpu.async_copy(tile_v.at[s, cb],
                                 out_hbm.at[t, cb, pl.ds(p * 2, 2)],
                                 osems[s])

        def wait_out(s):
            for cb in range(_D // 8):
                pltpu.make_async_copy(tile_v.at[s, cb],
                                      out_hbm.at[0, cb, pl.ds(0, 2)],
                                      osems[s]).wait()

        def transpose(s):
            rows = rows_v.at[s]
            for bg in range(_UB // 16):
                ridx = iota + (bg * 16)
                for c in range(_D):
                    val = plsc.load_gather(rows, [ridx, jnp.full((16,), c,
                                                                 jnp.int32)])
                    tile_v[s, c // 8, bg // 8, c % 8,
                           pl.ds((bg % 8) * 16, 16)] = val

        start_idx(0, u0)
        start_idx(1, u0 + 1)
        wait_idx(0)
        start_gather(0)

        def body(k, carry):
            s = lax.rem(k, 2)
            s1 = 1 - s

            @pl.when(k < per_w - 1)
            def _():
                for sv in range(_NBUF):
                    @pl.when(s1 == sv)
                    def _():
                        wait_idx(sv)
                        start_gather(sv)

            for sv in range(_NBUF):
                @pl.when(s == sv)
                def _():
                    @pl.when(k >= 2)
                    def _():
                        wait_out(sv)
                    wait_gather(sv)
                    transpose(sv)
                    start_out(sv, u0 + k)

                    @pl.when(k < per_w - 2)
                    def _():
                        start_idx(sv, u0 + k + 2)
            return carry

        lax.fori_loop(0, per_w, body, 0)
        wait_out(0)
        wait_out(1)

    return run


def kernel(input, weight):
    idx_flat = jnp.transpose(input).reshape(-1).astype(jnp.int32)
    out5d = _gather_fn(weight.shape[0])(idx_flat, weight)
    return jnp.transpose(out5d, (2, 4, 0, 1, 3)).reshape(_B, _T, _D)


# speedup vs baseline: 1.0312x; 1.0312x over previous
import functools

import jax
import jax.numpy as jnp
from jax import lax
from jax.experimental import pallas as pl
from jax.experimental.pallas import tpu as pltpu
from jax.experimental.pallas import tpu_sc as plsc

_T = 200
_B = 16384
_D = 32
_UB = 256
_NBUF = 2
_NV = _UB * _D // 16


def _gather_fn(v):
    info = plsc.get_sparse_core_info()
    nc, ns = info.num_cores, info.num_subcores
    nw = nc * ns
    nunits = _T * (_B // _UB)
    per_w = nunits // nw
    ppt = _B // _UB
    assert nunits % nw == 0

    mesh = plsc.VectorSubcoreMesh(core_axis_name="c", subcore_axis_name="s")

    @functools.partial(
        pl.kernel,
        out_type=jax.ShapeDtypeStruct((_T, _D // 8, (_B // 128) * 1024),
                                      jnp.float32),
        mesh=mesh,
        scratch_types=[
            pltpu.VMEM((_NBUF, _UB), jnp.int32),
            pltpu.VMEM((_NBUF, _UB, _D), jnp.float32),
            pltpu.VMEM((_NBUF, _UB * _D), jnp.float32),
            pltpu.VMEM((_NV, 16), jnp.int32),
            [pltpu.SemaphoreType.DMA] * _NBUF,
            [pltpu.SemaphoreType.DMA] * _NBUF,
            [pltpu.SemaphoreType.DMA] * _NBUF,
        ],
        compiler_params=pltpu.CompilerParams(use_tc_tiling_on_sc=False,
                                             needs_layout_passes=False),
    )
    def run(idx_hbm, table_hbm, out_hbm, idx_v, rows_v, tile_v, idxtab,
            isems, gsems, osems):
        wid = lax.axis_index("s") * nc + lax.axis_index("c")
        u0 = wid * per_w
        iota = lax.iota(jnp.int32, 16)

        def gen(k, carry):
            b = k >> 1
            h = k & 1
            c = h * 16 + iota
            idxtab[k, :] = ((c >> 3) * 2048 + (b >> 7) * 1024
                            + (c & 7) * 128 + (b & 127))
            return carry

        lax.fori_loop(0, _NV, gen, 0)

        def unit_tp(u):
            t = u // ppt
            p = u - t * ppt
            return t, p

        def start_idx(s, u):
            t, p = unit_tp(u)
            off = pl.multiple_of(t * _B + p * _UB, 8)
            pltpu.async_copy(idx_hbm.at[pl.ds(off, _UB)], idx_v.at[s],
                             isems[s])

        def wait_idx(s):
            pltpu.make_async_copy(idx_hbm.at[pl.ds(0, _UB)], idx_v.at[s],
                                  isems[s]).wait()

        def start_gather(s):
            pltpu.async_copy(table_hbm.at[idx_v.at[s]], rows_v.at[s],
                             gsems[s])

        def wait_gather(s):
            pltpu.make_async_copy(table_hbm.at[idx_v.at[s]], rows_v.at[s],
                                  gsems[s]).wait()

        def start_out(s, u):
            t, p = unit_tp(u)
            for cb in range(_D // 8):
                pltpu.async_copy(tile_v.at[s, pl.ds(cb * 2048, 2048)],
                                 out_hbm.at[t, cb, pl.ds(p * 2048, 2048)],
                                 osems[s])

        def wait_out(s):
            for cb in range(_D // 8):
                pltpu.make_async_copy(tile_v.at[s, pl.ds(cb * 2048, 2048)],
                                      out_hbm.at[0, cb, pl.ds(0, 2048)],
                                      osems[s]).wait()

        def transpose(s):
            rows = rows_v.at[s]
            for k in range(_NV):
                val = rows[k >> 1, pl.ds((k & 1) * 16, 16)]
                plsc.store_scatter(tile_v.at[s], [idxtab[k, :]], val)

        start_idx(0, u0)
        start_idx(1, u0 + 1)
        wait_idx(0)
        start_gather(0)

        def body(k, carry):
            s = lax.rem(k, 2)
            s1 = 1 - s

            @pl.when(k < per_w - 1)
            def _():
                for sv in range(_NBUF):
                    @pl.when(s1 == sv)
                    def _():
                        wait_idx(sv)
                        start_gather(sv)

            for sv in range(_NBUF):
                @pl.when(s == sv)
                def _():
                    @pl.when(k >= 2)
                    def _():
                        wait_out(sv)
                    wait_gather(sv)
                    transpose(sv)
                    start_out(sv, u0 + k)

                    @pl.when(k < per_w - 2)
                    def _():
                        start_idx(sv, u0 + k + 2)
            return carry

        lax.fori_loop(0, per_w, body, 0)
        wait_out(0)
        wait_out(1)

    return run


def kernel(input, weight):
    idx_flat = jnp.transpose(input).reshape(-1).astype(jnp.int32)
    out3d = _gather_fn(weight.shape[0])(idx_flat, weight)
    out5d = out3d.reshape(_T, _D // 8, _B // 128, 8, 128)
    return jnp.transpose(out5d, (2, 4, 0, 1, 3)).reshape(_B, _T, _D)


# speedup vs baseline: 1.0495x; 1.0178x over previous
import functools

import jax
import jax.numpy as jnp
from jax import lax
from jax.experimental import pallas as pl
from jax.experimental.pallas import tpu as pltpu
from jax.experimental.pallas import tpu_sc as plsc

_T = 200
_B = 16384
_D = 32
_UB = 256
_NBUF = 2
_NV = _UB * _D // 16


def _gather_fn(v):
    info = plsc.get_sparse_core_info()
    nc, ns = info.num_cores, info.num_subcores
    nw = nc * ns
    nunits = _T * (_B // _UB)
    per_w = nunits // nw
    ppt = _B // _UB
    assert nunits % nw == 0 and per_w % _NBUF == 0

    mesh = plsc.VectorSubcoreMesh(core_axis_name="c", subcore_axis_name="s")

    @functools.partial(
        pl.kernel,
        out_type=jax.ShapeDtypeStruct((_T, _D // 8, (_B // 128) * 1024),
                                      jnp.float32),
        mesh=mesh,
        scratch_types=[
            pltpu.VMEM((_NBUF, _UB), jnp.int32),
            pltpu.VMEM((_NBUF, _UB, _D), jnp.float32),
            pltpu.VMEM((_NBUF, _D // 8, (_UB // 128) * 1024), jnp.float32),
            pltpu.VMEM((_NV, 16), jnp.int32),
            [pltpu.SemaphoreType.DMA] * _NBUF,
            [pltpu.SemaphoreType.DMA] * _NBUF,
            [pltpu.SemaphoreType.DMA] * _NBUF,
        ],
        compiler_params=pltpu.CompilerParams(use_tc_tiling_on_sc=False,
                                             needs_layout_passes=False),
    )
    def run(idx_hbm, table_hbm, out_hbm, idx_v, rows_v, tile_v, idxtab,
            isems, gsems, osems):
        wid = lax.axis_index("s") * nc + lax.axis_index("c")
        u0 = wid * per_w
        iota = lax.iota(jnp.int32, 16)

        def gen(k, carry):
            b = k >> 1
            h = k & 1
            c = h * 16 + iota
            idxtab[k, :] = ((c >> 3) * 2048 + (b >> 7) * 1024
                            + (c & 7) * 128 + (b & 127))
            return carry

        lax.fori_loop(0, _NV, gen, 0)

        def unit_tp(u):
            t = u // ppt
            p = u - t * ppt
            return t, p

        def start_idx(s, u):
            t, p = unit_tp(u)
            off = pl.multiple_of(t * _B + p * _UB, 8)
            pltpu.async_copy(idx_hbm.at[pl.ds(off, _UB)], idx_v.at[s],
                             isems[s])

        def wait_idx(s):
            pltpu.make_async_copy(idx_hbm.at[pl.ds(0, _UB)], idx_v.at[s],
                                  isems[s]).wait()

        def start_gather(s):
            pltpu.async_copy(table_hbm.at[idx_v.at[s]], rows_v.at[s],
                             gsems[s])

        def wait_gather(s):
            pltpu.make_async_copy(table_hbm.at[idx_v.at[s]], rows_v.at[s],
                                  gsems[s]).wait()

        def start_out(s, u):
            t, p = unit_tp(u)
            pltpu.async_copy(tile_v.at[s],
                             out_hbm.at[t, :, pl.ds(p * 2048, 2048)],
                             osems[s])

        def wait_out(s):
            pltpu.make_async_copy(tile_v.at[s],
                                  out_hbm.at[0, :, pl.ds(0, 2048)],
                                  osems[s]).wait()

        def transpose(s):
            rows = rows_v.at[s]
            tiles = tile_v.at[s]

            def tbody(j, carry):
                for q in range(8):
                    b = j * 8 + q
                    for h in range(2):
                        val = rows[b, pl.ds(h * 16, 16)]
                        w = idxtab[b * 2 + h, :]
                        plsc.store_scatter(tiles, [w >> 11, w & 2047], val)
                return carry

            lax.fori_loop(0, _UB // 8, tbody, 0)

        start_idx(0, u0)
        start_idx(1, u0 + 1)
        wait_idx(0)
        start_gather(0)

        def body(k2, carry):
            for s in range(_NBUF):
                k = k2 * _NBUF + s
                s1 = 1 - s

                @pl.when(k < per_w - 1)
                def _():
                    wait_idx(s1)
                    start_gather(s1)

                @pl.when(k >= 2)
                def _():
                    wait_out(s)
                wait_gather(s)
                transpose(s)
                start_out(s, u0 + k)

                @pl.when(k < per_w - 2)
                def _():
                    start_idx(s, u0 + k + 2)
            return carry

        lax.fori_loop(0, per_w // _NBUF, body, 0)
        wait_out(0)
        wait_out(1)

    return run


def kernel(input, weight):
    idx_flat = jnp.transpose(input).reshape(-1).astype(jnp.int32)
    out3d = _gather_fn(weight.shape[0])(idx_flat, weight)
    out5d = out3d.reshape(_T, _D // 8, _B // 128, 8, 128)
    return jnp.transpose(out5d, (2, 4, 0, 1, 3)).reshape(_B, _T, _D)


# speedup vs baseline: 1.1336x; 1.0801x over previous
import functools

import jax
import jax.numpy as jnp
from jax import lax
from jax.experimental import pallas as pl
from jax.experimental.pallas import tpu as pltpu
from jax.experimental.pallas import tpu_sc as plsc

_T = 200
_B = 16384
_D = 32
_UB = 256
_NBUF = 2
_NV = _UB * _D // 16


def _gather_fn(v):
    info = plsc.get_sparse_core_info()
    nc, ns = info.num_cores, info.num_subcores
    nw = nc * ns
    nunits = _T * (_B // _UB)
    per_w = nunits // nw
    ppt = _B // _UB
    assert nunits % nw == 0 and per_w % _NBUF == 0

    mesh = plsc.VectorSubcoreMesh(core_axis_name="c", subcore_axis_name="s")

    @functools.partial(
        pl.kernel,
        out_type=jax.ShapeDtypeStruct((_T, _D // 8, (_B // 128) * 1024),
                                      jnp.float32),
        mesh=mesh,
        scratch_types=[
            pltpu.VMEM((_NBUF, _UB), jnp.int32),
            pltpu.VMEM((_NBUF, _UB, _D), jnp.float32),
            pltpu.VMEM((_NBUF, _D // 8, (_UB // 128) * 1024), jnp.float32),
            pltpu.VMEM((32, 16), jnp.int32),
            pltpu.VMEM((32, 16), jnp.int32),
            [pltpu.SemaphoreType.DMA] * _NBUF,
            [pltpu.SemaphoreType.DMA] * _NBUF,
            [pltpu.SemaphoreType.DMA] * _NBUF,
        ],
        compiler_params=pltpu.CompilerParams(use_tc_tiling_on_sc=False,
                                             needs_layout_passes=False),
    )
    def run(idx_hbm, table_hbm, out_hbm, idx_v, rows_v, tile_v, rpat, wpat,
            isems, gsems, osems):
        wid = lax.axis_index("s") * nc + lax.axis_index("c")
        u0 = wid * per_w
        iota = lax.iota(jnp.int32, 16)

        def gen(j, carry):
            i = j & 15
            h = j >> 4
            c = h * 16 + ((i + iota) & 15)
            rpat[j, :] = c
            wpat[j, :] = (c >> 3) * 2048 + (c & 7) * 128 + iota
            return carry

        lax.fori_loop(0, 32, gen, 0)

        def unit_tp(u):
            t = u // ppt
            p = u - t * ppt
            return t, p

        def start_idx(s, u):
            t, p = unit_tp(u)
            off = pl.multiple_of(t * _B + p * _UB, 8)
            pltpu.async_copy(idx_hbm.at[pl.ds(off, _UB)], idx_v.at[s],
                             isems[s])

        def wait_idx(s):
            pltpu.make_async_copy(idx_hbm.at[pl.ds(0, _UB)], idx_v.at[s],
                                  isems[s]).wait()

        def start_gather(s):
            pltpu.async_copy(table_hbm.at[idx_v.at[s]], rows_v.at[s],
                             gsems[s])

        def wait_gather(s):
            pltpu.make_async_copy(table_hbm.at[idx_v.at[s]], rows_v.at[s],
                                  gsems[s]).wait()

        def start_out(s, u):
            t, p = unit_tp(u)
            pltpu.async_copy(tile_v.at[s],
                             out_hbm.at[t, :, pl.ds(p * 2048, 2048)],
                             osems[s])

        def wait_out(s):
            pltpu.make_async_copy(tile_v.at[s],
                                  out_hbm.at[0, :, pl.ds(0, 2048)],
                                  osems[s]).wait()

        def transpose(s):
            rows = rows_v.at[s]
            tiles = tile_v.at[s]

            def tbody(bg, carry):
                b0 = bg * 16
                brow = iota + b0
                wbase = (bg >> 3) * 1024 + (bg & 7) * 16
                for j in range(32):
                    val = plsc.load_gather(rows, [brow, rpat[j, :]])
                    w = wpat[j, :] + wbase
                    plsc.store_scatter(tiles, [w >> 11, w & 2047], val)
                return carry

            lax.fori_loop(0, _UB // 16, tbody, 0)

        start_idx(0, u0)
        start_idx(1, u0 + 1)
        wait_idx(0)
        start_gather(0)

        def body(k2, carry):
            for s in range(_NBUF):
                k = k2 * _NBUF + s
                s1 = 1 - s

                @pl.when(k < per_w - 1)
                def _():
                    wait_idx(s1)
                    start_gather(s1)

                @pl.when(k >= 2)
                def _():
                    wait_out(s)
                wait_gather(s)
                transpose(s)
                start_out(s, u0 + k)

                @pl.when(k < per_w - 2)
                def _():
                    start_idx(s, u0 + k + 2)
            return carry

        lax.fori_loop(0, per_w // _NBUF, body, 0)
        wait_out(0)
        wait_out(1)

    return run


def kernel(input, weight):
    idx_flat = jnp.transpose(input).reshape(-1).astype(jnp.int32)
    out3d = _gather_fn(weight.shape[0])(idx_flat, weight)
    out5d = out3d.reshape(_T, _D // 8, _B // 128, 8, 128)
    return jnp.transpose(out5d, (2, 4, 0, 1, 3)).reshape(_B, _T, _D)


# speedup vs baseline: 2.4004x; 2.1175x over previous
import functools

import jax
import jax.numpy as jnp
from jax import lax
from jax.experimental import pallas as pl
from jax.experimental.pallas import tpu as pltpu
from jax.experimental.pallas import tpu_sc as plsc

_T = 200
_B = 16384
_D = 32
_UB = 512
_NG = 4
_NO = 2
_PL = (_UB // 128) * 1024


def _gather_fn(v):
    info = plsc.get_sparse_core_info()
    nc, ns = info.num_cores, info.num_subcores
    nw = nc * ns
    nunits = _T * (_B // _UB)
    per_w = nunits // nw
    ppt = _B // _UB
    assert nunits % nw == 0 and per_w % _NG == 0

    mesh = plsc.VectorSubcoreMesh(core_axis_name="c", subcore_axis_name="s")

    @functools.partial(
        pl.kernel,
        out_type=jax.ShapeDtypeStruct((_T, _D // 8, (_B // 128) * 1024),
                                      jnp.float32),
        mesh=mesh,
        scratch_types=[
            pltpu.VMEM((_NG, _UB), jnp.int32),
            pltpu.VMEM((_NG, _UB, _D), jnp.float32),
            pltpu.VMEM((_NO, _D // 8, _PL), jnp.float32),
            [pltpu.SemaphoreType.DMA] * _NG,
            [pltpu.SemaphoreType.DMA] * _NG,
            [pltpu.SemaphoreType.DMA] * _NO,
        ],
        compiler_params=pltpu.CompilerParams(use_tc_tiling_on_sc=False,
                                             needs_layout_passes=False),
    )
    def run(idx_hbm, table_hbm, out_hbm, idx_v, rows_v, tile_v,
            isems, gsems, osems):
        wid = lax.axis_index("s") * nc + lax.axis_index("c")
        u0 = wid * per_w
        iota = lax.iota(jnp.int32, 16)

        rots = [(iota + i) & 15 for i in range(16)]
        wps = [(r >> 3) * _PL + (r & 7) * 128 + iota for r in rots]

        def unit_tp(u):
            t = u // ppt
            p = u - t * ppt
            return t, p

        def start_idx(s, u):
            t, p = unit_tp(u)
            off = pl.multiple_of(t * _B + p * _UB, 8)
            pltpu.async_copy(idx_hbm.at[pl.ds(off, _UB)], idx_v.at[s],
                             isems[s])

        def wait_idx(s):
            pltpu.make_async_copy(idx_hbm.at[pl.ds(0, _UB)], idx_v.at[s],
                                  isems[s]).wait()

        def start_gather(s):
            pltpu.async_copy(table_hbm.at[idx_v.at[s]], rows_v.at[s],
                             gsems[s])

        def wait_gather(s):
            pltpu.make_async_copy(table_hbm.at[idx_v.at[s]], rows_v.at[s],
                                  gsems[s]).wait()

        def start_out(s, u):
            t, p = unit_tp(u)
            pltpu.async_copy(tile_v.at[s],
                             out_hbm.at[t, :, pl.ds(p * _PL, _PL)],
                             osems[s])

        def wait_out(s):
            pltpu.make_async_copy(tile_v.at[s],
                                  out_hbm.at[0, :, pl.ds(0, _PL)],
                                  osems[s]).wait()

        def transpose(rs, ts):
            rows = rows_v.at[rs]
            tiles = tile_v.at[ts]

            def tbody(bg, carry):
                brow = iota + bg * 16
                wb0 = (bg >> 3) * 1024 + (bg & 7) * 16
                for h in range(2):
                    wbase = wb0 + h * 2 * _PL
                    for i in range(16):
                        val = plsc.load_gather(rows,
                                               [brow, rots[i] + h * 16])
                        w = wps[i] + wbase
                        plsc.store_scatter(
                            tiles, [w // _PL, lax.rem(w, _PL)], val)
                return carry

            lax.fori_loop(0, _UB // 16, tbody, 0)

        for s in range(_NG):
            start_idx(s, u0 + s)
        wait_idx(0)
        start_gather(0)
        wait_idx(1)
        start_gather(1)

        def body(k4, carry):
            for sv in range(_NG):
                k = k4 * _NG + sv
                ts = sv % _NO
                wait_gather(sv)

                @pl.when(k + 2 < per_w)
                def _():
                    s2 = (sv + 2) % _NG
                    wait_idx(s2)
                    start_gather(s2)

                @pl.when(k >= _NO)
                def _():
                    wait_out(ts)
                transpose(sv, ts)
                start_out(ts, u0 + k)

                @pl.when(k + _NG < per_w)
                def _():
                    start_idx(sv, u0 + k + _NG)
            return carry

        lax.fori_loop(0, per_w // _NG, body, 0)
        for s in range(_NO):
            wait_out(s)

    return run


def kernel(input, weight):
    idx_flat = jnp.transpose(input).reshape(-1).astype(jnp.int32)
    out3d = _gather_fn(weight.shape[0])(idx_flat, weight)
    out5d = out3d.reshape(_T, _D // 8, _B // 128, 8, 128)
    return jnp.transpose(out5d, (2, 4, 0, 1, 3)).reshape(_B, _T, _D)
